# baseline (speedup 1.0000x reference)
import functools

import jax
import jax.numpy as jnp
from jax import lax
from jax.experimental import pallas as pl
from jax.experimental.pallas import tpu as pltpu
from jax.experimental.pallas import tpu_sc as plsc

B, L, HIDDEN = 4096, 200, 128
N = B * L
NC, NS = 2, 16
NW = NC * NS
LANES = 16
MAX_POS, TYPE_VOCAB = 512, 2
CROWS = MAX_POS * TYPE_VOCAB
NBUF = 3


def _build(n_tokens, hidden, k_chunk):
    tpw = n_tokens // NW
    chunks = tpw // k_chunk
    quads = chunks // NBUF
    kr = k_chunk // LANES
    ng = hidden // (2 * LANES)
    rows = n_tokens // LANES
    mesh = plsc.VectorSubcoreMesh(
        core_axis_name="c", subcore_axis_name="s", num_cores=NC, num_subcores=NS
    )

    @functools.partial(
        pl.kernel,
        out_type=jax.ShapeDtypeStruct((rows, LANES, hidden), jnp.float32),
        mesh=mesh,
        scratch_types=[
            pltpu.VMEM((NBUF, kr, 2, LANES), jnp.int32),
            pltpu.VMEM((NBUF, kr, LANES, hidden), jnp.float32),
            pltpu.VMEM((CROWS * hidden // 2,), jnp.int32),
            [pltpu.SemaphoreType.DMA] * NBUF,
            [pltpu.SemaphoreType.DMA] * NBUF,
            [pltpu.SemaphoreType.DMA] * NBUF,
        ],
    )
    def sc_embed(idx2_hbm, wt_hbm, ct_hbm, out_hbm, idx, wb, ctv,
                 semg, semw, semi):
        wid = lax.axis_index("s") * NC + lax.axis_index("c")
        base0 = wid * (tpw // LANES)

        pltpu.sync_copy(ct_hbm, ctv)

        def fire_idx(i, b):
            pltpu.async_copy(
                idx2_hbm.at[pl.ds(base0 + i * kr, kr)], idx.at[b], semi[b]
            )

        def fire_gather(i, b):
            for j in range(kr):
                pltpu.async_copy(wt_hbm.at[idx.at[b, j, 0]], wb.at[b, j], semg[b])

        def drain_idx(b):
            pltpu.make_async_copy(
                idx2_hbm.at[pl.ds(0, kr)], idx.at[b], semi[b]
            ).wait()

        def drain_gather(b):
            pltpu.make_async_copy(
                out_hbm.at[pl.ds(0, kr)], wb.at[b], semg[b]
            ).wait()

        def drain_wb(b):
            pltpu.make_async_copy(
                wb.at[b], out_hbm.at[pl.ds(0, kr)], semw[b]
            ).wait()

        for j in range(NBUF):
            fire_idx(j, j)
        for j in range(NBUF - 1):
            drain_idx(j)
            fire_gather(j, j)

        peel = chunks % NBUF
        rounds = (chunks - peel) // NBUF

        def process(i, b, b3):
            drain_gather(b)

            @plsc.parallel_loop(0, kr, 1, unroll=2)
            def grp_body(r):
                cvec = idx[b, r, 1, :]

                def loads(q):
                    cc = cvec[q]
                    return [
                        ctv[pl.ds(cc + g * LANES, LANES)]
                        for g in range(ng)
                    ]

                prev = loads(0)
                for q in range(LANES):
                    nxt = loads(q + 1) if q + 1 < LANES else None
                    for g in range(ng):
                        xi = prev[g]
                        va = lax.bitcast_convert_type(xi << 16, jnp.float32)
                        vb = lax.bitcast_convert_type(
                            xi & jnp.int32(-65536), jnp.float32
                        )
                        col = g * 2 * LANES
                        plsc.addupdate(
                            wb.at[b, r, q, pl.ds(col, LANES)], va
                        )
                        plsc.addupdate(
                            wb.at[b, r, q, pl.ds(col + LANES, LANES)], vb
                        )
                    prev = nxt

            pltpu.async_copy(
                wb.at[b], out_hbm.at[pl.ds(base0 + i * kr, kr)], semw[b]
            )

            @pl.when(i + (NBUF - 1) < chunks)
            def _():
                @pl.when(i >= 1)
                def _():
                    drain_wb(b3)

                drain_idx(b3)
                fire_gather(i + (NBUF - 1), b3)

            @pl.when(i + NBUF < chunks)
            def _():
                fire_idx(i + NBUF, b)

        def round_body(h, carry):
            i0 = NBUF * h
            for c in range(NBUF):
                process(i0 + c, c, (c + NBUF - 1) % NBUF)
            return carry

        lax.fori_loop(0, rounds, round_body, 0, unroll=False)
        for j in range(peel):
            i = chunks - peel + j
            process(i, i % NBUF, (i + NBUF - 1) % NBUF)
        for j in range(NBUF):
            drain_wb(j)

    return sc_embed


def kernel(input_ids, position_ids, token_type_ids, word_table, pos_table, seg_table):
    ids = input_ids.reshape(N // LANES, LANES).astype(jnp.int32)
    cix = (
        (position_ids * TYPE_VOCAB + token_type_ids).reshape(N // LANES, LANES)
        * (HIDDEN // 2)
    ).astype(jnp.int32)
    idx2 = jnp.stack([ids, cix], axis=1)
    ctab = (pos_table[:, None, :] + seg_table[None, :, :]).reshape(CROWS, HIDDEN)
    ctab = (
        ctab.reshape(CROWS, HIDDEN // 32, 2, LANES)
        .transpose(0, 1, 3, 2)
        .astype(jnp.bfloat16)
    )
    ctab = jax.lax.bitcast_convert_type(ctab, jnp.int32).reshape(-1)
    fn = _build(N, HIDDEN, 128)
    out = fn(idx2, word_table, ctab)
    return out.reshape(B, L, HIDDEN)


# speedup vs baseline: 1.2026x; 1.2026x over previous
import functools

import jax
import jax.numpy as jnp
from jax import lax
from jax.experimental import pallas as pl
from jax.experimental.pallas import tpu as pltpu
from jax.experimental.pallas import tpu_sc as plsc

B, L, HIDDEN = 4096, 200, 128
N = B * L
NC, NS = 2, 16
NW = NC * NS
LANES = 16
MAX_POS, TYPE_VOCAB = 512, 2
CROWS = MAX_POS * TYPE_VOCAB
NBUF = 3


def _build(n_tokens, hidden, k_chunk):
    tpw = n_tokens // NW
    chunks = tpw // k_chunk
    quads = chunks // NBUF
    kr = k_chunk // LANES
    ng = hidden // (2 * LANES)
    rows = n_tokens // LANES
    mesh = plsc.VectorSubcoreMesh(
        core_axis_name="c", subcore_axis_name="s", num_cores=NC, num_subcores=NS
    )

    @functools.partial(
        pl.kernel,
        out_type=jax.ShapeDtypeStruct((n_tokens, hidden), jnp.float32),
        mesh=mesh,
        scratch_types=[
            pltpu.VMEM((NBUF, 2, k_chunk), jnp.int32),
            pltpu.VMEM((NBUF, k_chunk, hidden), jnp.float32),
            pltpu.VMEM((CROWS * hidden // 2,), jnp.int32),
            [pltpu.SemaphoreType.DMA] * NBUF,
            [pltpu.SemaphoreType.DMA] * NBUF,
            [pltpu.SemaphoreType.DMA] * NBUF,
        ],
    )
    def sc_embed(idx2_hbm, wt_hbm, ct_hbm, out_hbm, idx, wb, ctv,
                 semg, semw, semi):
        wid = lax.axis_index("s") * NC + lax.axis_index("c")
        base0 = wid * tpw
        cbase0 = wid * chunks

        pltpu.sync_copy(ct_hbm, ctv)

        def fire_idx(i, b):
            pltpu.async_copy(idx2_hbm.at[cbase0 + i], idx.at[b], semi[b])

        def fire_gather(i, b):
            pltpu.async_copy(wt_hbm.at[idx.at[b, 0]], wb.at[b], semg[b])

        def drain_idx(b):
            pltpu.make_async_copy(idx2_hbm.at[0], idx.at[b], semi[b]).wait()

        def drain_gather(b):
            pltpu.make_async_copy(
                out_hbm.at[pl.ds(0, k_chunk)], wb.at[b], semg[b]
            ).wait()

        def drain_wb(b):
            pltpu.make_async_copy(
                wb.at[b], out_hbm.at[pl.ds(0, k_chunk)], semw[b]
            ).wait()

        for j in range(NBUF):
            fire_idx(j, j)
        for j in range(NBUF - 1):
            drain_idx(j)
            fire_gather(j, j)

        peel = chunks % NBUF
        rounds = (chunks - peel) // NBUF

        def process(i, b, b3):
            drain_gather(b)

            @plsc.parallel_loop(0, kr, 1, unroll=2)
            def grp_body(r):
                cvec = idx[b, 1, pl.ds(r * LANES, LANES)]

                def loads(q):
                    cc = cvec[q]
                    return [
                        ctv[pl.ds(cc + g * LANES, LANES)]
                        for g in range(ng)
                    ]

                prev = loads(0)
                for q in range(LANES):
                    nxt = loads(q + 1) if q + 1 < LANES else None
                    for g in range(ng):
                        xi = prev[g]
                        va = lax.bitcast_convert_type(xi << 16, jnp.float32)
                        vb = lax.bitcast_convert_type(
                            xi & jnp.int32(-65536), jnp.float32
                        )
                        col = g * 2 * LANES
                        t = r * LANES + q
                        plsc.addupdate(
                            wb.at[b, t, pl.ds(col, LANES)], va
                        )
                        plsc.addupdate(
                            wb.at[b, t, pl.ds(col + LANES, LANES)], vb
                        )
                    prev = nxt

            pltpu.async_copy(
                wb.at[b], out_hbm.at[pl.ds(base0 + i * k_chunk, k_chunk)], semw[b]
            )

            @pl.when(i + (NBUF - 1) < chunks)
            def _():
                @pl.when(i >= 1)
                def _():
                    drain_wb(b3)

                drain_idx(b3)
                fire_gather(i + (NBUF - 1), b3)

            @pl.when(i + NBUF < chunks)
            def _():
                fire_idx(i + NBUF, b)

        def round_body(h, carry):
            i0 = NBUF * h
            for c in range(NBUF):
                process(i0 + c, c, (c + NBUF - 1) % NBUF)
            return carry

        lax.fori_loop(0, rounds, round_body, 0, unroll=False)
        for j in range(peel):
            i = chunks - peel + j
            process(i, i % NBUF, (i + NBUF - 1) % NBUF)
        for j in range(NBUF):
            drain_wb(j)

    return sc_embed


def kernel(input_ids, position_ids, token_type_ids, word_table, pos_table, seg_table):
    K = 128
    ids = input_ids.reshape(N // K, K).astype(jnp.int32)
    cix = (
        (position_ids * TYPE_VOCAB + token_type_ids).reshape(N // K, K)
        * (HIDDEN // 2)
    ).astype(jnp.int32)
    idx2 = jnp.stack([ids, cix], axis=1)
    ctab = (pos_table[:, None, :] + seg_table[None, :, :]).reshape(CROWS, HIDDEN)
    ctab = (
        ctab.reshape(CROWS, HIDDEN // 32, 2, LANES)
        .transpose(0, 1, 3, 2)
        .astype(jnp.bfloat16)
    )
    ctab = jax.lax.bitcast_convert_type(ctab, jnp.int32).reshape(-1)
    fn = _build(N, HIDDEN, 128)
    out = fn(idx2, word_table, ctab)
    return out.reshape(B, L, HIDDEN)
